# baseline (speedup 1.0000x reference)
import jax
import jax.numpy as jnp
from jax import lax
from jax.experimental import pallas as pl
from jax.experimental.pallas import tpu as pltpu
from jax.experimental.pallas import tpu_sc as plsc

N_NODES = 10000
N_EDGES = 320000
D = 128

NC = 2
NS = 16
NW = NC * NS
N_PAD = 10240
ROWS_PER_TILE = N_PAD // NS

CHUNK = 64
E_PER_WORKER = 10240
NCH = E_PER_WORKER // CHUNK
E_PADDED = NW * E_PER_WORKER
N_EDGE_PAD = E_PADDED - N_EDGES

DEG_CHUNK = 80
DEG_NCH = E_PADDED // NW // DEG_CHUNK
N_DEG = 10240
DEG_PER_TILE = N_DEG // NS

ROW_BLK = 400
N_ROW_BLKS = N_NODES // ROW_BLK


def _zero_vmem_2d(ref, rows, cols):
    z = jnp.zeros((16,), jnp.float32)
    cl = cols // 16

    def body(i, carry):
        r = i // cl
        c = (i % cl) * 16
        ref[r, pl.ds(c, 16)] = z
        return carry

    lax.fori_loop(0, rows * cl, body, 0)


def _zero_vmem_1d(ref, n):
    z = jnp.zeros((16,), jnp.float32)

    def body(i, carry):
        ref[pl.ds(i * 16, 16)] = z
        return carry

    lax.fori_loop(0, n // 16, body, 0)


def _degree_body(dst_hbm, out_hbm, acc, idx_d, ones_v, zeros_v):
    cid = lax.axis_index("c")
    sid = lax.axis_index("s")
    wid = sid * NC + cid

    one = jnp.ones((16,), jnp.float32)

    def fill(i, carry):
        ones_v[pl.ds(i * 16, 16)] = one
        return carry

    lax.fori_loop(0, DEG_CHUNK // 16, fill, 0)
    _zero_vmem_1d(zeros_v, DEG_PER_TILE)
    pltpu.sync_copy(zeros_v, acc.at[pl.ds(sid * DEG_PER_TILE, DEG_PER_TILE)])
    plsc.subcore_barrier()

    pltpu.sync_copy(dst_hbm.at[wid], idx_d)

    def body(j, carry):
        pltpu.sync_copy(ones_v, acc.at[idx_d.at[j]], add=True)
        return carry

    lax.fori_loop(0, DEG_NCH, body, 0)
    plsc.subcore_barrier()

    sl = pl.ds(sid * DEG_PER_TILE, DEG_PER_TILE)
    pltpu.sync_copy(acc.at[sl], out_hbm.at[cid, sl])


def _degree_partials(dst3):
    mesh = plsc.VectorSubcoreMesh(core_axis_name="c", subcore_axis_name="s")
    return pl.kernel(
        _degree_body,
        out_type=jax.ShapeDtypeStruct((NC, N_DEG), jnp.float32),
        mesh=mesh,
        scratch_types=[
            pltpu.VMEM_SHARED((N_DEG,), jnp.float32),
            pltpu.VMEM((DEG_NCH, DEG_CHUNK), jnp.int32),
            pltpu.VMEM((DEG_CHUNK,), jnp.float32),
            pltpu.VMEM((DEG_PER_TILE,), jnp.float32),
        ],
    )(dst3)


def _aggregate_body(y_hbm, src_hbm, dst_hbm, out_hbm, acc, idx_s, idx_d,
                    b0, b1, b2, b3, s0, s1, s2, s3, t0, t1, t2, t3):
    cid = lax.axis_index("c")
    sid = lax.axis_index("s")
    wid = sid * NC + cid

    _zero_vmem_2d(b0, CHUNK, D)
    base = sid * ROWS_PER_TILE

    def zero_acc(t, carry):
        pltpu.sync_copy(b0, acc.at[pl.ds(base + t * CHUNK, CHUNK)])
        return carry

    lax.fori_loop(0, ROWS_PER_TILE // CHUNK, zero_acc, 0)
    plsc.subcore_barrier()

    pltpu.sync_copy(src_hbm.at[wid], idx_s)
    pltpu.sync_copy(dst_hbm.at[wid], idx_d)

    def gather(j, buf, sem):
        sl = pl.ds(j * CHUNK, CHUNK)
        pltpu.async_copy(y_hbm.at[idx_s.at[sl]], buf, sem)

    def wait_gather(j, buf, sem):
        sl = pl.ds(j * CHUNK, CHUNK)
        pltpu.make_async_copy(y_hbm.at[idx_s.at[sl]], buf, sem).wait()

    def scatter(j, buf, sem):
        sl = pl.ds(j * CHUNK, CHUNK)
        pltpu.async_copy(buf, acc.at[idx_d.at[sl]], sem, add=True)

    def wait_scatter(j, buf, sem):
        sl = pl.ds(j * CHUNK, CHUNK)
        pltpu.make_async_copy(buf, acc.at[idx_d.at[sl]], sem).wait()

    B = (b0, b1, b2)
    SG = (s0, s1, s2)
    SS = (t0, t1, t2)

    gather(0, b0, s0)
    gather(1, b1, s1)
    wait_gather(0, b0, s0)
    scatter(0, b0, t0)
    gather(2, b2, s2)
    wait_gather(1, b1, s1)
    scatter(1, b1, t1)
    wait_scatter(0, b0, t0)
    gather(3, b0, s0)
    wait_gather(2, b2, s2)
    scatter(2, b2, t2)
    wait_scatter(1, b1, t1)
    gather(4, b1, s1)

    def step(j, k):
        kg = (k + 2) % 3
        wait_gather(j, B[k], SG[k])
        scatter(j, B[k], SS[k])
        wait_scatter(j - 1, B[kg], SS[kg])
        gather(j + 2, B[kg], SG[kg])

    def body(t, carry):
        j = 3 * t
        step(j, 0)
        step(j + 1, 1)
        step(j + 2, 2)
        return carry

    lax.fori_loop(1, (NCH - 4) // 3, body, 0)
    j = NCH - 4
    step(j, 0)
    step(j + 1, 1)
    wait_gather(j + 2, b2, s2)
    scatter(j + 2, b2, t2)
    wait_scatter(j + 1, b1, t1)
    wait_gather(j + 3, b0, s0)
    scatter(j + 3, b0, t0)
    wait_scatter(j + 2, b2, t2)
    wait_scatter(j + 3, b0, t0)
    plsc.subcore_barrier()

    sl = pl.ds(base, ROWS_PER_TILE)
    pltpu.sync_copy(acc.at[sl], out_hbm.at[cid, sl])


def _aggregate_partials(y, srcT, dstT):
    mesh = plsc.VectorSubcoreMesh(core_axis_name="c", subcore_axis_name="s")
    return pl.kernel(
        _aggregate_body,
        out_type=jax.ShapeDtypeStruct((NC, N_PAD, D), jnp.float32),
        mesh=mesh,
        scratch_types=[
            pltpu.VMEM_SHARED((N_PAD, D), jnp.float32),
            pltpu.VMEM((E_PER_WORKER,), jnp.int32),
            pltpu.VMEM((E_PER_WORKER,), jnp.int32),
            pltpu.VMEM((CHUNK, D), jnp.float32),
            pltpu.VMEM((CHUNK, D), jnp.float32),
            pltpu.VMEM((CHUNK, D), jnp.float32),
            pltpu.VMEM((CHUNK, D), jnp.float32),
            pltpu.SemaphoreType.DMA,
            pltpu.SemaphoreType.DMA,
            pltpu.SemaphoreType.DMA,
            pltpu.SemaphoreType.DMA,
            pltpu.SemaphoreType.DMA,
            pltpu.SemaphoreType.DMA,
            pltpu.SemaphoreType.DMA,
            pltpu.SemaphoreType.DMA,
        ],
    )(y, srcT, dstT)


def _norm_from_deg(d_ref):
    deg = d_ref[0] + d_ref[1]
    return lax.rsqrt(jnp.maximum(deg, 1.0))


def _pre_body(x_ref, w_ref, d_ref, o_ref):
    norm = _norm_from_deg(d_ref)
    xw = jnp.dot(x_ref[...], w_ref[...], preferred_element_type=jnp.float32)
    o_ref[...] = xw * norm


def _mid_body(p_ref, w_ref, b_ref, d_ref, o_ref):
    norm = _norm_from_deg(d_ref)
    b = b_ref[0:1, :]
    h = jnp.maximum((p_ref[0] + p_ref[1]) * norm + b, 0.0)
    hw = jnp.dot(h, w_ref[...], preferred_element_type=jnp.float32)
    o_ref[...] = hw * norm


def _post_body(p_ref, b_ref, d_ref, o_ref):
    norm = _norm_from_deg(d_ref)
    b = b_ref[0:1, :]
    o_ref[...] = jnp.maximum((p_ref[0] + p_ref[1]) * norm + b, 0.0)


def _row_spec():
    return pl.BlockSpec((ROW_BLK, D), lambda i: (i, 0))


def _part_spec():
    return pl.BlockSpec((2, ROW_BLK, D), lambda i: (0, i, 0))


def _deg_spec():
    return pl.BlockSpec((2, ROW_BLK, 1), lambda i: (0, i, 0))


def _full_spec(shape):
    return pl.BlockSpec(shape, lambda i: tuple(0 for _ in shape))


def _tc_pre(x, w, deg3):
    return pl.pallas_call(
        _pre_body,
        grid=(N_ROW_BLKS,),
        in_specs=[_row_spec(), _full_spec((D, D)), _deg_spec()],
        out_specs=_row_spec(),
        out_shape=jax.ShapeDtypeStruct((N_NODES, D), jnp.float32),
    )(x, w, deg3)


def _tc_mid(p, w, b8, deg3):
    return pl.pallas_call(
        _mid_body,
        grid=(N_ROW_BLKS,),
        in_specs=[_part_spec(), _full_spec((D, D)), _full_spec((8, D)),
                  _deg_spec()],
        out_specs=_row_spec(),
        out_shape=jax.ShapeDtypeStruct((N_NODES, D), jnp.float32),
    )(p, w, b8, deg3)


def _tc_post(p, b8, deg3):
    return pl.pallas_call(
        _post_body,
        grid=(N_ROW_BLKS,),
        in_specs=[_part_spec(), _full_spec((8, D)), _deg_spec()],
        out_specs=_row_spec(),
        out_shape=jax.ShapeDtypeStruct((N_NODES, D), jnp.float32),
    )(p, b8, deg3)


def kernel(features, edge_index, W1, b1, W2, b2):
    pad_src = (jnp.arange(N_EDGE_PAD, dtype=jnp.int32) * 131) % N_NODES
    pad_dst = N_NODES + (jnp.arange(N_EDGE_PAD, dtype=jnp.int32)
                         % (N_PAD - N_NODES))
    srcp = jnp.concatenate([edge_index[0], pad_src])
    dstp = jnp.concatenate([edge_index[1], pad_dst])

    srcT = srcp.reshape(NW, E_PER_WORKER)
    dstT = dstp.reshape(NW, E_PER_WORKER)
    dst3 = dstp.reshape(NW, DEG_NCH, DEG_CHUNK)

    b1_8 = jnp.broadcast_to(b1[None, :], (8, D))
    b2_8 = jnp.broadcast_to(b2[None, :], (8, D))

    deg_p = _degree_partials(dst3)
    deg3 = deg_p.reshape(NC, N_DEG, 1)

    y1 = _tc_pre(features, W1, deg3)
    agg1 = _aggregate_partials(y1, srcT, dstT)
    y2 = _tc_mid(agg1, W2, b1_8, deg3)
    agg2 = _aggregate_partials(y2, srcT, dstT)
    return _tc_post(agg2, b2_8, deg3)


# speedup vs baseline: 1.0716x; 1.0716x over previous
import jax
import jax.numpy as jnp
from jax import lax
from jax.experimental import pallas as pl
from jax.experimental.pallas import tpu as pltpu
from jax.experimental.pallas import tpu_sc as plsc

N_NODES = 10000
N_EDGES = 320000
D = 128

NC = 2
NS = 16
NW = NC * NS
N_PAD = 10240
ROWS_PER_TILE = N_PAD // NS

CHUNK = 64
E_PER_WORKER = 10240
BANK = E_PER_WORKER // 2
NB = BANK // CHUNK
E_PADDED = NW * E_PER_WORKER
N_EDGE_PAD = E_PADDED - N_EDGES

DEG_CHUNK = 80
DEG_NCH = E_PADDED // NW // DEG_CHUNK
N_DEG = 10240
DEG_PER_TILE = N_DEG // NS

ROW_BLK = 400
N_ROW_BLKS = N_NODES // ROW_BLK


def _zero_vmem_2d(ref, rows, cols):
    z = jnp.zeros((16,), jnp.float32)
    cl = cols // 16

    def body(i, carry):
        r = i // cl
        c = (i % cl) * 16
        ref[r, pl.ds(c, 16)] = z
        return carry

    lax.fori_loop(0, rows * cl, body, 0)


def _zero_vmem_1d(ref, n):
    z = jnp.zeros((16,), jnp.float32)

    def body(i, carry):
        ref[pl.ds(i * 16, 16)] = z
        return carry

    lax.fori_loop(0, n // 16, body, 0)


def _degree_body(dst_hbm, out_hbm, acc, idx_d, ones_v, zeros_v):
    cid = lax.axis_index("c")
    sid = lax.axis_index("s")
    wid = sid * NC + cid

    one = jnp.ones((16,), jnp.float32)

    def fill(i, carry):
        ones_v[pl.ds(i * 16, 16)] = one
        return carry

    lax.fori_loop(0, DEG_CHUNK // 16, fill, 0)
    _zero_vmem_1d(zeros_v, DEG_PER_TILE)
    pltpu.sync_copy(zeros_v, acc.at[pl.ds(sid * DEG_PER_TILE, DEG_PER_TILE)])
    plsc.subcore_barrier()

    pltpu.sync_copy(dst_hbm.at[wid], idx_d)

    def body(j, carry):
        pltpu.sync_copy(ones_v, acc.at[idx_d.at[j]], add=True)
        return carry

    lax.fori_loop(0, DEG_NCH, body, 0)
    plsc.subcore_barrier()

    sl = pl.ds(sid * DEG_PER_TILE, DEG_PER_TILE)
    pltpu.sync_copy(acc.at[sl], out_hbm.at[cid, sl])


def _degree_partials(dst3):
    mesh = plsc.VectorSubcoreMesh(core_axis_name="c", subcore_axis_name="s")
    return pl.kernel(
        _degree_body,
        out_type=jax.ShapeDtypeStruct((NC, N_DEG), jnp.float32),
        mesh=mesh,
        scratch_types=[
            pltpu.VMEM_SHARED((N_DEG,), jnp.float32),
            pltpu.VMEM((DEG_NCH, DEG_CHUNK), jnp.int32),
            pltpu.VMEM((DEG_CHUNK,), jnp.float32),
            pltpu.VMEM((DEG_PER_TILE,), jnp.float32),
        ],
    )(dst3)


def _aggregate_body(y_hbm, src_hbm, dst_hbm, out_hbm, acc, idx_s, idx_d,
                    b0, b1, b2, b3, s0, s1, s2, s3):
    cid = lax.axis_index("c")
    sid = lax.axis_index("s")
    wid = sid * NC + cid

    _zero_vmem_2d(b0, CHUNK, D)
    base = sid * ROWS_PER_TILE

    def zero_acc(t, carry):
        pltpu.sync_copy(b0, acc.at[pl.ds(base + t * CHUNK, CHUNK)])
        return carry

    lax.fori_loop(0, ROWS_PER_TILE // CHUNK, zero_acc, 0)
    plsc.subcore_barrier()

    def gather(j, buf, sem):
        sl = pl.ds(j * CHUNK, CHUNK)
        pltpu.async_copy(y_hbm.at[idx_s.at[sl]], buf, sem)

    def wait_gather(j, buf, sem):
        sl = pl.ds(j * CHUNK, CHUNK)
        pltpu.make_async_copy(y_hbm.at[idx_s.at[sl]], buf, sem).wait()

    def scatter(j, buf):
        sl = pl.ds(j * CHUNK, CHUNK)
        pltpu.sync_copy(buf, acc.at[idx_d.at[sl]], add=True)

    def stage(bank):
        pltpu.sync_copy(src_hbm.at[wid, bank], idx_s)
        pltpu.sync_copy(dst_hbm.at[wid, bank], idx_d)

    def run_bank():
        gather(0, b0, s0)
        gather(1, b1, s1)
        gather(2, b2, s2)

        def body(t, carry):
            j = 4 * t
            gather(j + 3, b3, s3)
            wait_gather(j, b0, s0)
            scatter(j, b0)
            gather(j + 4, b0, s0)
            wait_gather(j + 1, b1, s1)
            scatter(j + 1, b1)
            gather(j + 5, b1, s1)
            wait_gather(j + 2, b2, s2)
            scatter(j + 2, b2)
            gather(j + 6, b2, s2)
            wait_gather(j + 3, b3, s3)
            scatter(j + 3, b3)
            return carry

        lax.fori_loop(0, NB // 4 - 1, body, 0)
        j = NB - 4
        gather(j + 3, b3, s3)
        wait_gather(j, b0, s0)
        scatter(j, b0)
        wait_gather(j + 1, b1, s1)
        scatter(j + 1, b1)
        wait_gather(j + 2, b2, s2)
        scatter(j + 2, b2)
        wait_gather(j + 3, b3, s3)
        scatter(j + 3, b3)

    stage(0)
    run_bank()
    stage(1)
    run_bank()
    plsc.subcore_barrier()

    sl = pl.ds(base, ROWS_PER_TILE)
    pltpu.sync_copy(acc.at[sl], out_hbm.at[cid, sl])


def _aggregate_partials(y, srcT, dstT):
    mesh = plsc.VectorSubcoreMesh(core_axis_name="c", subcore_axis_name="s")
    return pl.kernel(
        _aggregate_body,
        out_type=jax.ShapeDtypeStruct((NC, N_PAD, D), jnp.float32),
        mesh=mesh,
        scratch_types=[
            pltpu.VMEM_SHARED((N_PAD, D), jnp.float32),
            pltpu.VMEM((BANK,), jnp.int32),
            pltpu.VMEM((BANK,), jnp.int32),
            pltpu.VMEM((CHUNK, D), jnp.float32),
            pltpu.VMEM((CHUNK, D), jnp.float32),
            pltpu.VMEM((CHUNK, D), jnp.float32),
            pltpu.VMEM((CHUNK, D), jnp.float32),
            pltpu.SemaphoreType.DMA,
            pltpu.SemaphoreType.DMA,
            pltpu.SemaphoreType.DMA,
            pltpu.SemaphoreType.DMA,
        ],
    )(y, srcT, dstT)


def _norm_from_deg(d_ref):
    deg = d_ref[0] + d_ref[1]
    return lax.rsqrt(jnp.maximum(deg, 1.0))


def _pre_body(x_ref, w_ref, d_ref, o_ref):
    norm = _norm_from_deg(d_ref)
    xw = jnp.dot(x_ref[...], w_ref[...], preferred_element_type=jnp.float32)
    o_ref[...] = xw * norm


def _mid_body(p_ref, w_ref, b_ref, d_ref, o_ref):
    norm = _norm_from_deg(d_ref)
    b = b_ref[0:1, :]
    h = jnp.maximum((p_ref[0] + p_ref[1]) * norm + b, 0.0)
    hw = jnp.dot(h, w_ref[...], preferred_element_type=jnp.float32)
    o_ref[...] = hw * norm


def _post_body(p_ref, b_ref, d_ref, o_ref):
    norm = _norm_from_deg(d_ref)
    b = b_ref[0:1, :]
    o_ref[...] = jnp.maximum((p_ref[0] + p_ref[1]) * norm + b, 0.0)


def _row_spec():
    return pl.BlockSpec((ROW_BLK, D), lambda i: (i, 0))


def _part_spec():
    return pl.BlockSpec((2, ROW_BLK, D), lambda i: (0, i, 0))


def _deg_spec():
    return pl.BlockSpec((2, ROW_BLK, 1), lambda i: (0, i, 0))


def _full_spec(shape):
    return pl.BlockSpec(shape, lambda i: tuple(0 for _ in shape))


def _tc_pre(x, w, deg3):
    return pl.pallas_call(
        _pre_body,
        grid=(N_ROW_BLKS,),
        in_specs=[_row_spec(), _full_spec((D, D)), _deg_spec()],
        out_specs=_row_spec(),
        out_shape=jax.ShapeDtypeStruct((N_NODES, D), jnp.float32),
    )(x, w, deg3)


def _tc_mid(p, w, b8, deg3):
    return pl.pallas_call(
        _mid_body,
        grid=(N_ROW_BLKS,),
        in_specs=[_part_spec(), _full_spec((D, D)), _full_spec((8, D)),
                  _deg_spec()],
        out_specs=_row_spec(),
        out_shape=jax.ShapeDtypeStruct((N_NODES, D), jnp.float32),
    )(p, w, b8, deg3)


def _tc_post(p, b8, deg3):
    return pl.pallas_call(
        _post_body,
        grid=(N_ROW_BLKS,),
        in_specs=[_part_spec(), _full_spec((8, D)), _deg_spec()],
        out_specs=_row_spec(),
        out_shape=jax.ShapeDtypeStruct((N_NODES, D), jnp.float32),
    )(p, b8, deg3)


def kernel(features, edge_index, W1, b1, W2, b2):
    pad_src = (jnp.arange(N_EDGE_PAD, dtype=jnp.int32) * 131) % N_NODES
    pad_dst = N_NODES + (jnp.arange(N_EDGE_PAD, dtype=jnp.int32)
                         % (N_PAD - N_NODES))
    srcp = jnp.concatenate([edge_index[0], pad_src])
    dstp = jnp.concatenate([edge_index[1], pad_dst])

    srcT = srcp.reshape(NW, 2, BANK)
    dstT = dstp.reshape(NW, 2, BANK)
    dst3 = dstp.reshape(NW, DEG_NCH, DEG_CHUNK)

    b1_8 = jnp.broadcast_to(b1[None, :], (8, D))
    b2_8 = jnp.broadcast_to(b2[None, :], (8, D))

    deg_p = _degree_partials(dst3)
    deg3 = deg_p.reshape(NC, N_DEG, 1)

    y1 = _tc_pre(features, W1, deg3)
    agg1 = _aggregate_partials(y1, srcT, dstT)
    y2 = _tc_mid(agg1, W2, b1_8, deg3)
    agg2 = _aggregate_partials(y2, srcT, dstT)
    return _tc_post(agg2, b2_8, deg3)


# speedup vs baseline: 1.1848x; 1.1056x over previous
import jax
import jax.numpy as jnp
from jax import lax
from jax.experimental import pallas as pl
from jax.experimental.pallas import tpu as pltpu
from jax.experimental.pallas import tpu_sc as plsc

N_NODES = 10000
N_EDGES = 320000
D = 128

NC = 2
NS = 16
NW = NC * NS
N_PAD = 10240
ROWS_PER_TILE = N_PAD // NS

CHUNK = 64
E_PER_WORKER = 10240
BANK = E_PER_WORKER // 2
NB = BANK // CHUNK
E_PADDED = NW * E_PER_WORKER
N_EDGE_PAD = E_PADDED - N_EDGES

DEG_CHUNK = 80
DEG_NCH = E_PADDED // NW // DEG_CHUNK
N_DEG = 10240
DEG_PER_TILE = N_DEG // NS

ROW_BLK = 2000
N_ROW_BLKS = N_NODES // ROW_BLK


def _zero_vmem_2d(ref, rows, cols):
    z = jnp.zeros((16,), jnp.float32)
    cl = cols // 16

    def body(i, carry):
        r = i // cl
        c = (i % cl) * 16
        ref[r, pl.ds(c, 16)] = z
        return carry

    lax.fori_loop(0, rows * cl, body, 0)


def _zero_vmem_1d(ref, n):
    z = jnp.zeros((16,), jnp.float32)

    def body(i, carry):
        ref[pl.ds(i * 16, 16)] = z
        return carry

    lax.fori_loop(0, n // 16, body, 0)


def _degree_body(dst_hbm, out_hbm, acc, idx_d, ones_v, zeros_v):
    cid = lax.axis_index("c")
    sid = lax.axis_index("s")
    wid = sid * NC + cid

    one = jnp.ones((16,), jnp.float32)

    def fill(i, carry):
        ones_v[pl.ds(i * 16, 16)] = one
        return carry

    lax.fori_loop(0, DEG_CHUNK // 16, fill, 0)
    _zero_vmem_1d(zeros_v, DEG_PER_TILE)
    pltpu.sync_copy(zeros_v, acc.at[pl.ds(sid * DEG_PER_TILE, DEG_PER_TILE)])
    plsc.subcore_barrier()

    pltpu.sync_copy(dst_hbm.at[wid], idx_d)

    def body(j, carry):
        pltpu.sync_copy(ones_v, acc.at[idx_d.at[j]], add=True)
        return carry

    lax.fori_loop(0, DEG_NCH, body, 0)
    plsc.subcore_barrier()

    sl = pl.ds(sid * DEG_PER_TILE, DEG_PER_TILE)
    pltpu.sync_copy(acc.at[sl], out_hbm.at[cid, sl])


def _degree_partials(dst3):
    mesh = plsc.VectorSubcoreMesh(core_axis_name="c", subcore_axis_name="s")
    return pl.kernel(
        _degree_body,
        out_type=jax.ShapeDtypeStruct((NC, N_DEG), jnp.float32),
        mesh=mesh,
        scratch_types=[
            pltpu.VMEM_SHARED((N_DEG,), jnp.float32),
            pltpu.VMEM((DEG_NCH, DEG_CHUNK), jnp.int32),
            pltpu.VMEM((DEG_CHUNK,), jnp.float32),
            pltpu.VMEM((DEG_PER_TILE,), jnp.float32),
        ],
    )(dst3)


def _aggregate_body(y_hbm, src_hbm, dst_hbm, out_hbm, acc, idx_s, idx_d,
                    b0, b1, b2, b3, s0, s1, s2, s3):
    cid = lax.axis_index("c")
    sid = lax.axis_index("s")
    wid = sid * NC + cid

    _zero_vmem_2d(b0, CHUNK, D)
    base = sid * ROWS_PER_TILE

    def zero_acc(t, carry):
        pltpu.sync_copy(b0, acc.at[pl.ds(base + t * CHUNK, CHUNK)])
        return carry

    lax.fori_loop(0, ROWS_PER_TILE // CHUNK, zero_acc, 0)
    plsc.subcore_barrier()

    def gather(j, buf, sem):
        sl = pl.ds(j * CHUNK, CHUNK)
        pltpu.async_copy(y_hbm.at[idx_s.at[sl]], buf, sem)

    def wait_gather(j, buf, sem):
        sl = pl.ds(j * CHUNK, CHUNK)
        pltpu.make_async_copy(y_hbm.at[idx_s.at[sl]], buf, sem).wait()

    def scatter(j, buf):
        sl = pl.ds(j * CHUNK, CHUNK)
        pltpu.sync_copy(buf, acc.at[idx_d.at[sl]], add=True)

    def stage(bank):
        pltpu.sync_copy(src_hbm.at[wid, bank], idx_s)
        pltpu.sync_copy(dst_hbm.at[wid, bank], idx_d)

    def run_bank():
        gather(0, b0, s0)
        gather(1, b1, s1)
        gather(2, b2, s2)

        def body(t, carry):
            j = 4 * t
            gather(j + 3, b3, s3)
            wait_gather(j, b0, s0)
            scatter(j, b0)
            gather(j + 4, b0, s0)
            wait_gather(j + 1, b1, s1)
            scatter(j + 1, b1)
            gather(j + 5, b1, s1)
            wait_gather(j + 2, b2, s2)
            scatter(j + 2, b2)
            gather(j + 6, b2, s2)
            wait_gather(j + 3, b3, s3)
            scatter(j + 3, b3)
            return carry

        lax.fori_loop(0, NB // 4 - 1, body, 0)
        j = NB - 4
        gather(j + 3, b3, s3)
        wait_gather(j, b0, s0)
        scatter(j, b0)
        wait_gather(j + 1, b1, s1)
        scatter(j + 1, b1)
        wait_gather(j + 2, b2, s2)
        scatter(j + 2, b2)
        wait_gather(j + 3, b3, s3)
        scatter(j + 3, b3)

    stage(0)
    run_bank()
    stage(1)
    run_bank()
    plsc.subcore_barrier()

    sl = pl.ds(base, ROWS_PER_TILE)
    pltpu.sync_copy(acc.at[sl], out_hbm.at[cid, sl])


def _aggregate_partials(y, srcT, dstT):
    mesh = plsc.VectorSubcoreMesh(core_axis_name="c", subcore_axis_name="s")
    return pl.kernel(
        _aggregate_body,
        out_type=jax.ShapeDtypeStruct((NC, N_PAD, D), jnp.float32),
        mesh=mesh,
        scratch_types=[
            pltpu.VMEM_SHARED((N_PAD, D), jnp.float32),
            pltpu.VMEM((BANK,), jnp.int32),
            pltpu.VMEM((BANK,), jnp.int32),
            pltpu.VMEM((CHUNK, D), jnp.float32),
            pltpu.VMEM((CHUNK, D), jnp.float32),
            pltpu.VMEM((CHUNK, D), jnp.float32),
            pltpu.VMEM((CHUNK, D), jnp.float32),
            pltpu.SemaphoreType.DMA,
            pltpu.SemaphoreType.DMA,
            pltpu.SemaphoreType.DMA,
            pltpu.SemaphoreType.DMA,
        ],
    )(y, srcT, dstT)


def _norm_from_deg(d_ref):
    deg = d_ref[0] + d_ref[1]
    return lax.rsqrt(jnp.maximum(deg, 1.0))


def _pre_body(x_ref, w_ref, d_ref, o_ref):
    norm = _norm_from_deg(d_ref)
    xw = jnp.dot(x_ref[...], w_ref[...], preferred_element_type=jnp.float32)
    o_ref[...] = xw * norm


def _mid_body(p_ref, w_ref, b_ref, d_ref, o_ref):
    norm = _norm_from_deg(d_ref)
    b = b_ref[0:1, :]
    h = jnp.maximum((p_ref[0] + p_ref[1]) * norm + b, 0.0)
    hw = jnp.dot(h, w_ref[...], preferred_element_type=jnp.float32)
    o_ref[...] = hw * norm


def _post_body(p_ref, b_ref, d_ref, o_ref):
    norm = _norm_from_deg(d_ref)
    b = b_ref[0:1, :]
    o_ref[...] = jnp.maximum((p_ref[0] + p_ref[1]) * norm + b, 0.0)


def _row_spec():
    return pl.BlockSpec((ROW_BLK, D), lambda i: (i, 0))


def _part_spec():
    return pl.BlockSpec((2, ROW_BLK, D), lambda i: (0, i, 0))


def _deg_spec():
    return pl.BlockSpec((2, ROW_BLK, 1), lambda i: (0, i, 0))


def _full_spec(shape):
    return pl.BlockSpec(shape, lambda i: tuple(0 for _ in shape))


def _tc_pre(x, w, deg3):
    return pl.pallas_call(
        _pre_body,
        grid=(N_ROW_BLKS,),
        in_specs=[_row_spec(), _full_spec((D, D)), _deg_spec()],
        out_specs=_row_spec(),
        out_shape=jax.ShapeDtypeStruct((N_NODES, D), jnp.float32),
    )(x, w, deg3)


def _tc_mid(p, w, b8, deg3):
    return pl.pallas_call(
        _mid_body,
        grid=(N_ROW_BLKS,),
        in_specs=[_part_spec(), _full_spec((D, D)), _full_spec((8, D)),
                  _deg_spec()],
        out_specs=_row_spec(),
        out_shape=jax.ShapeDtypeStruct((N_NODES, D), jnp.float32),
    )(p, w, b8, deg3)


def _tc_post(p, b8, deg3):
    return pl.pallas_call(
        _post_body,
        grid=(N_ROW_BLKS,),
        in_specs=[_part_spec(), _full_spec((8, D)), _deg_spec()],
        out_specs=_row_spec(),
        out_shape=jax.ShapeDtypeStruct((N_NODES, D), jnp.float32),
    )(p, b8, deg3)


def kernel(features, edge_index, W1, b1, W2, b2):
    pad_src = (jnp.arange(N_EDGE_PAD, dtype=jnp.int32) * 131) % N_NODES
    pad_dst = N_NODES + (jnp.arange(N_EDGE_PAD, dtype=jnp.int32)
                         % (N_PAD - N_NODES))
    srcp = jnp.concatenate([edge_index[0], pad_src])
    dstp = jnp.concatenate([edge_index[1], pad_dst])

    srcT = srcp.reshape(NW, 2, BANK)
    dstT = dstp.reshape(NW, 2, BANK)
    dst3 = dstp.reshape(NW, DEG_NCH, DEG_CHUNK)

    b1_8 = jnp.broadcast_to(b1[None, :], (8, D))
    b2_8 = jnp.broadcast_to(b2[None, :], (8, D))

    deg_p = _degree_partials(dst3)
    deg3 = deg_p.reshape(NC, N_DEG, 1)

    y1 = _tc_pre(features, W1, deg3)
    agg1 = _aggregate_partials(y1, srcT, dstT)
    y2 = _tc_mid(agg1, W2, b1_8, deg3)
    agg2 = _aggregate_partials(y2, srcT, dstT)
    return _tc_post(agg2, b2_8, deg3)


# speedup vs baseline: 1.1900x; 1.0044x over previous
import jax
import jax.numpy as jnp
from jax import lax
from jax.experimental import pallas as pl
from jax.experimental.pallas import tpu as pltpu
from jax.experimental.pallas import tpu_sc as plsc

N_NODES = 10000
N_EDGES = 320000
D = 128

NC = 2
NS = 16
NW = NC * NS
N_PAD = 10240
ROWS_PER_TILE = N_PAD // NS

CHUNK = 64
E_PER_WORKER = 10240
E_LAST = N_EDGES - 31 * E_PER_WORKER
BANK = E_PER_WORKER // 2
BANK_LAST = E_LAST // 2
NB = BANK // CHUNK
NB_LAST = BANK_LAST // CHUNK

DEG_CHUNK = 80
DEG_PER_WORKER = N_EDGES // NW
DEG_NCH = DEG_PER_WORKER // DEG_CHUNK
N_DEG = 10240
DEG_PER_TILE = N_DEG // NS

ROW_BLK = 2000
N_ROW_BLKS = N_NODES // ROW_BLK


def _zero_vmem_2d(ref, rows, cols):
    z = jnp.zeros((16,), jnp.float32)
    cl = cols // 16

    def body(i, carry):
        r = i // cl
        c = (i % cl) * 16
        ref[r, pl.ds(c, 16)] = z
        return carry

    lax.fori_loop(0, rows * cl, body, 0)


def _zero_vmem_1d(ref, n):
    z = jnp.zeros((16,), jnp.float32)

    def body(i, carry):
        ref[pl.ds(i * 16, 16)] = z
        return carry

    lax.fori_loop(0, n // 16, body, 0)


def _degree_body(dst_hbm, out_hbm, acc, idx_d, ones_v, zeros_v):
    cid = lax.axis_index("c")
    sid = lax.axis_index("s")
    wid = sid * NC + cid

    one = jnp.ones((16,), jnp.float32)

    def fill(i, carry):
        ones_v[pl.ds(i * 16, 16)] = one
        return carry

    lax.fori_loop(0, DEG_CHUNK // 16, fill, 0)
    _zero_vmem_1d(zeros_v, DEG_PER_TILE)
    pltpu.sync_copy(zeros_v, acc.at[pl.ds(sid * DEG_PER_TILE, DEG_PER_TILE)])
    plsc.subcore_barrier()

    pltpu.sync_copy(dst_hbm.at[pl.ds(wid * DEG_PER_WORKER, DEG_PER_WORKER)],
                    idx_d)

    def body(j, carry):
        pltpu.sync_copy(ones_v, acc.at[idx_d.at[pl.ds(j * DEG_CHUNK,
                                                      DEG_CHUNK)]], add=True)
        return carry

    lax.fori_loop(0, DEG_NCH, body, 0)
    plsc.subcore_barrier()

    sl = pl.ds(sid * DEG_PER_TILE, DEG_PER_TILE)
    pltpu.sync_copy(acc.at[sl], out_hbm.at[cid, sl])


def _degree_partials(dst):
    mesh = plsc.VectorSubcoreMesh(core_axis_name="c", subcore_axis_name="s")
    return pl.kernel(
        _degree_body,
        out_type=jax.ShapeDtypeStruct((NC, N_DEG), jnp.float32),
        mesh=mesh,
        scratch_types=[
            pltpu.VMEM_SHARED((N_DEG,), jnp.float32),
            pltpu.VMEM((DEG_PER_WORKER,), jnp.int32),
            pltpu.VMEM((DEG_CHUNK,), jnp.float32),
            pltpu.VMEM((DEG_PER_TILE,), jnp.float32),
        ],
    )(dst)


def _aggregate_body(y_hbm, src_hbm, dst_hbm, out_hbm, acc, idx_s, idx_d,
                    b0, b1, b2, b3, s0, s1, s2, s3):
    cid = lax.axis_index("c")
    sid = lax.axis_index("s")
    wid = sid * NC + cid

    _zero_vmem_2d(b0, CHUNK, D)
    base = sid * ROWS_PER_TILE

    def zero_acc(t, carry):
        pltpu.sync_copy(b0, acc.at[pl.ds(base + t * CHUNK, CHUNK)])
        return carry

    lax.fori_loop(0, ROWS_PER_TILE // CHUNK, zero_acc, 0)
    plsc.subcore_barrier()

    def gather(j, buf, sem):
        sl = pl.ds(j * CHUNK, CHUNK)
        pltpu.async_copy(y_hbm.at[idx_s.at[sl]], buf, sem)

    def wait_gather(j, buf, sem):
        sl = pl.ds(j * CHUNK, CHUNK)
        pltpu.make_async_copy(y_hbm.at[idx_s.at[sl]], buf, sem).wait()

    def scatter(j, buf):
        sl = pl.ds(j * CHUNK, CHUNK)
        pltpu.sync_copy(buf, acc.at[idx_d.at[sl]], add=True)

    is_last = wid == NW - 1
    n_body = jnp.where(is_last, NB_LAST // 4 - 1, NB // 4 - 1)
    j_tail = jnp.where(is_last, NB_LAST - 4, NB - 4)

    def stage(bank):
        @pl.when(jnp.logical_not(is_last))
        def _():
            off = wid * E_PER_WORKER + bank * BANK
            pltpu.sync_copy(src_hbm.at[pl.ds(off, BANK)], idx_s)
            pltpu.sync_copy(dst_hbm.at[pl.ds(off, BANK)], idx_d)

        @pl.when(is_last)
        def _():
            off = (NW - 1) * E_PER_WORKER + bank * BANK_LAST
            pltpu.sync_copy(src_hbm.at[pl.ds(off, BANK_LAST)],
                            idx_s.at[pl.ds(0, BANK_LAST)])
            pltpu.sync_copy(dst_hbm.at[pl.ds(off, BANK_LAST)],
                            idx_d.at[pl.ds(0, BANK_LAST)])

    def run_bank():
        gather(0, b0, s0)
        gather(1, b1, s1)
        gather(2, b2, s2)

        def body(t, carry):
            j = 4 * t
            gather(j + 3, b3, s3)
            wait_gather(j, b0, s0)
            scatter(j, b0)
            gather(j + 4, b0, s0)
            wait_gather(j + 1, b1, s1)
            scatter(j + 1, b1)
            gather(j + 5, b1, s1)
            wait_gather(j + 2, b2, s2)
            scatter(j + 2, b2)
            gather(j + 6, b2, s2)
            wait_gather(j + 3, b3, s3)
            scatter(j + 3, b3)
            return carry

        lax.fori_loop(0, n_body, body, 0)
        j = j_tail
        gather(j + 3, b3, s3)
        wait_gather(j, b0, s0)
        scatter(j, b0)
        wait_gather(j + 1, b1, s1)
        scatter(j + 1, b1)
        wait_gather(j + 2, b2, s2)
        scatter(j + 2, b2)
        wait_gather(j + 3, b3, s3)
        scatter(j + 3, b3)

    stage(0)
    run_bank()
    stage(1)
    run_bank()
    plsc.subcore_barrier()

    sl = pl.ds(base, ROWS_PER_TILE)
    pltpu.sync_copy(acc.at[sl], out_hbm.at[cid, sl])


def _aggregate_partials(y, srcT, dstT):
    mesh = plsc.VectorSubcoreMesh(core_axis_name="c", subcore_axis_name="s")
    return pl.kernel(
        _aggregate_body,
        out_type=jax.ShapeDtypeStruct((NC, N_PAD, D), jnp.float32),
        mesh=mesh,
        scratch_types=[
            pltpu.VMEM_SHARED((N_PAD, D), jnp.float32),
            pltpu.VMEM((BANK,), jnp.int32),
            pltpu.VMEM((BANK,), jnp.int32),
            pltpu.VMEM((CHUNK, D), jnp.float32),
            pltpu.VMEM((CHUNK, D), jnp.float32),
            pltpu.VMEM((CHUNK, D), jnp.float32),
            pltpu.VMEM((CHUNK, D), jnp.float32),
            pltpu.SemaphoreType.DMA,
            pltpu.SemaphoreType.DMA,
            pltpu.SemaphoreType.DMA,
            pltpu.SemaphoreType.DMA,
        ],
    )(y, srcT, dstT)


def _norm_from_deg(d_ref):
    deg = d_ref[0] + d_ref[1]
    return lax.rsqrt(jnp.maximum(deg, 1.0))


def _pre_body(x_ref, w_ref, d_ref, o_ref):
    norm = _norm_from_deg(d_ref)
    xw = jnp.dot(x_ref[...], w_ref[...], preferred_element_type=jnp.float32)
    o_ref[...] = xw * norm


def _mid_body(p_ref, w_ref, b_ref, d_ref, o_ref):
    norm = _norm_from_deg(d_ref)
    b = b_ref[0:1, :]
    h = jnp.maximum((p_ref[0] + p_ref[1]) * norm + b, 0.0)
    hw = jnp.dot(h, w_ref[...], preferred_element_type=jnp.float32)
    o_ref[...] = hw * norm


def _post_body(p_ref, b_ref, d_ref, o_ref):
    norm = _norm_from_deg(d_ref)
    b = b_ref[0:1, :]
    o_ref[...] = jnp.maximum((p_ref[0] + p_ref[1]) * norm + b, 0.0)


def _row_spec():
    return pl.BlockSpec((ROW_BLK, D), lambda i: (i, 0))


def _part_spec():
    return pl.BlockSpec((2, ROW_BLK, D), lambda i: (0, i, 0))


def _deg_spec():
    return pl.BlockSpec((2, ROW_BLK, 1), lambda i: (0, i, 0))


def _full_spec(shape):
    return pl.BlockSpec(shape, lambda i: tuple(0 for _ in shape))


def _tc_pre(x, w, deg3):
    return pl.pallas_call(
        _pre_body,
        grid=(N_ROW_BLKS,),
        in_specs=[_row_spec(), _full_spec((D, D)), _deg_spec()],
        out_specs=_row_spec(),
        out_shape=jax.ShapeDtypeStruct((N_NODES, D), jnp.float32),
    )(x, w, deg3)


def _tc_mid(p, w, b8, deg3):
    return pl.pallas_call(
        _mid_body,
        grid=(N_ROW_BLKS,),
        in_specs=[_part_spec(), _full_spec((D, D)), _full_spec((8, D)),
                  _deg_spec()],
        out_specs=_row_spec(),
        out_shape=jax.ShapeDtypeStruct((N_NODES, D), jnp.float32),
    )(p, w, b8, deg3)


def _tc_post(p, b8, deg3):
    return pl.pallas_call(
        _post_body,
        grid=(N_ROW_BLKS,),
        in_specs=[_part_spec(), _full_spec((8, D)), _deg_spec()],
        out_specs=_row_spec(),
        out_shape=jax.ShapeDtypeStruct((N_NODES, D), jnp.float32),
    )(p, b8, deg3)


def kernel(features, edge_index, W1, b1, W2, b2):
    srcT = edge_index[0]
    dstT = edge_index[1]

    b1_8 = jnp.broadcast_to(b1[None, :], (8, D))
    b2_8 = jnp.broadcast_to(b2[None, :], (8, D))

    deg_p = _degree_partials(dstT)
    deg3 = deg_p.reshape(NC, N_DEG, 1)

    y1 = _tc_pre(features, W1, deg3)
    agg1 = _aggregate_partials(y1, srcT, dstT)
    y2 = _tc_mid(agg1, W2, b1_8, deg3)
    agg2 = _aggregate_partials(y2, srcT, dstT)
    return _tc_post(agg2, b2_8, deg3)


# speedup vs baseline: 1.3369x; 1.1234x over previous
import jax
import jax.numpy as jnp
from jax import lax
from jax.experimental import pallas as pl
from jax.experimental.pallas import tpu as pltpu
from jax.experimental.pallas import tpu_sc as plsc

N_NODES = 10000
N_EDGES = 320000
D = 128

NC = 2
NS = 16
NW = NC * NS
N_PAD = 10240
ROWS_PER_TILE = N_PAD // NS

CHUNK = 64
E_PER_WORKER = 10240
E_LAST = N_EDGES - 31 * E_PER_WORKER
BANK = E_PER_WORKER // 2
BANK_LAST = E_LAST // 2
NB = BANK // CHUNK
NB_LAST = BANK_LAST // CHUNK

DEG_CHUNK = 512
DEG_NCH = E_PER_WORKER // DEG_CHUNK
DEG_NCH_LAST = E_LAST // DEG_CHUNK
N_DEG = 10240
DEG_PER_TILE = N_DEG // NS

ROW_BLK = 2048
N_ROW_BLKS = -(-N_NODES // ROW_BLK)


def _zero_vmem_2d(ref, rows, cols):
    z = jnp.zeros((16,), jnp.float32)
    cl = cols // 16

    def body(i, carry):
        r = i // cl
        c = (i % cl) * 16
        ref[r, pl.ds(c, 16)] = z
        return carry

    lax.fori_loop(0, rows * cl, body, 0)


def _zero_vmem_1d(ref, n):
    z = jnp.zeros((16,), jnp.float32)

    def body(i, carry):
        ref[pl.ds(i * 16, 16)] = z
        return carry

    lax.fori_loop(0, n // 16, body, 0)


def _degree_body(ei_hbm, out_hbm, acc, idx_d, ones_v, zeros_v):
    cid = lax.axis_index("c")
    sid = lax.axis_index("s")
    wid = sid * NC + cid

    one = jnp.ones((16,), jnp.float32)

    def fill(i, carry):
        ones_v[pl.ds(i * 16, 16)] = one
        return carry

    lax.fori_loop(0, DEG_CHUNK // 16, fill, 0)
    _zero_vmem_1d(zeros_v, DEG_PER_TILE)
    pltpu.sync_copy(zeros_v, acc.at[pl.ds(sid * DEG_PER_TILE, DEG_PER_TILE)])
    plsc.subcore_barrier()

    is_last = wid == NW - 1

    @pl.when(jnp.logical_not(is_last))
    def _():
        off = N_EDGES + wid * E_PER_WORKER
        pltpu.sync_copy(ei_hbm.at[pl.ds(off, E_PER_WORKER)], idx_d)

    @pl.when(is_last)
    def _():
        off = N_EDGES + (NW - 1) * E_PER_WORKER
        pltpu.sync_copy(ei_hbm.at[pl.ds(off, E_LAST)],
                        idx_d.at[pl.ds(0, E_LAST)])

    def body(j, carry):
        pltpu.sync_copy(ones_v, acc.at[idx_d.at[pl.ds(j * DEG_CHUNK,
                                                      DEG_CHUNK)]], add=True)
        return carry

    lax.fori_loop(0, jnp.where(is_last, DEG_NCH_LAST, DEG_NCH), body, 0)
    plsc.subcore_barrier()

    sl = pl.ds(sid * DEG_PER_TILE, DEG_PER_TILE)
    pltpu.sync_copy(acc.at[sl], out_hbm.at[cid, sl])


def _degree_partials(ei):
    mesh = plsc.VectorSubcoreMesh(core_axis_name="c", subcore_axis_name="s")
    return pl.kernel(
        _degree_body,
        out_type=jax.ShapeDtypeStruct((NC, N_DEG), jnp.float32),
        mesh=mesh,
        scratch_types=[
            pltpu.VMEM_SHARED((N_DEG,), jnp.float32),
            pltpu.VMEM((E_PER_WORKER,), jnp.int32),
            pltpu.VMEM((DEG_CHUNK,), jnp.float32),
            pltpu.VMEM((DEG_PER_TILE,), jnp.float32),
        ],
    )(ei)


def _aggregate_body(y_hbm, ei_hbm, out_hbm, acc, idx_s, idx_d,
                    b0, b1, b2, b3, s0, s1, s2, s3):
    cid = lax.axis_index("c")
    sid = lax.axis_index("s")
    wid = sid * NC + cid

    _zero_vmem_2d(b0, CHUNK, D)
    base = sid * ROWS_PER_TILE

    def zero_acc(t, carry):
        pltpu.sync_copy(b0, acc.at[pl.ds(base + t * CHUNK, CHUNK)])
        return carry

    lax.fori_loop(0, ROWS_PER_TILE // CHUNK, zero_acc, 0)
    plsc.subcore_barrier()

    def gather(j, buf, sem):
        sl = pl.ds(j * CHUNK, CHUNK)
        pltpu.async_copy(y_hbm.at[idx_s.at[sl]], buf, sem)

    def wait_gather(j, buf, sem):
        sl = pl.ds(j * CHUNK, CHUNK)
        pltpu.make_async_copy(y_hbm.at[idx_s.at[sl]], buf, sem).wait()

    def scatter(j, buf):
        sl = pl.ds(j * CHUNK, CHUNK)
        pltpu.sync_copy(buf, acc.at[idx_d.at[sl]], add=True)

    is_last = wid == NW - 1
    n_body = jnp.where(is_last, NB_LAST // 4 - 1, NB // 4 - 1)
    j_tail = jnp.where(is_last, NB_LAST - 4, NB - 4)

    def stage(bank):
        @pl.when(jnp.logical_not(is_last))
        def _():
            off = wid * E_PER_WORKER + bank * BANK
            pltpu.sync_copy(ei_hbm.at[pl.ds(off, BANK)], idx_s)
            pltpu.sync_copy(ei_hbm.at[pl.ds(N_EDGES + off, BANK)], idx_d)

        @pl.when(is_last)
        def _():
            off = (NW - 1) * E_PER_WORKER + bank * BANK_LAST
            pltpu.sync_copy(ei_hbm.at[pl.ds(off, BANK_LAST)],
                            idx_s.at[pl.ds(0, BANK_LAST)])
            pltpu.sync_copy(ei_hbm.at[pl.ds(N_EDGES + off, BANK_LAST)],
                            idx_d.at[pl.ds(0, BANK_LAST)])

    def run_bank():
        gather(0, b0, s0)
        gather(1, b1, s1)
        gather(2, b2, s2)

        def body(t, carry):
            j = 4 * t
            gather(j + 3, b3, s3)
            wait_gather(j, b0, s0)
            scatter(j, b0)
            gather(j + 4, b0, s0)
            wait_gather(j + 1, b1, s1)
            scatter(j + 1, b1)
            gather(j + 5, b1, s1)
            wait_gather(j + 2, b2, s2)
            scatter(j + 2, b2)
            gather(j + 6, b2, s2)
            wait_gather(j + 3, b3, s3)
            scatter(j + 3, b3)
            return carry

        lax.fori_loop(0, n_body, body, 0)
        j = j_tail
        gather(j + 3, b3, s3)
        wait_gather(j, b0, s0)
        scatter(j, b0)
        wait_gather(j + 1, b1, s1)
        scatter(j + 1, b1)
        wait_gather(j + 2, b2, s2)
        scatter(j + 2, b2)
        wait_gather(j + 3, b3, s3)
        scatter(j + 3, b3)

    stage(0)
    run_bank()
    stage(1)
    run_bank()
    plsc.subcore_barrier()

    sl = pl.ds(base, ROWS_PER_TILE)
    pltpu.sync_copy(acc.at[sl], out_hbm.at[cid, sl])


def _aggregate_partials(y, ei):
    mesh = plsc.VectorSubcoreMesh(core_axis_name="c", subcore_axis_name="s")
    return pl.kernel(
        _aggregate_body,
        out_type=jax.ShapeDtypeStruct((NC, N_PAD, D), jnp.float32),
        mesh=mesh,
        scratch_types=[
            pltpu.VMEM_SHARED((N_PAD, D), jnp.float32),
            pltpu.VMEM((BANK,), jnp.int32),
            pltpu.VMEM((BANK,), jnp.int32),
            pltpu.VMEM((CHUNK, D), jnp.float32),
            pltpu.VMEM((CHUNK, D), jnp.float32),
            pltpu.VMEM((CHUNK, D), jnp.float32),
            pltpu.VMEM((CHUNK, D), jnp.float32),
            pltpu.SemaphoreType.DMA,
            pltpu.SemaphoreType.DMA,
            pltpu.SemaphoreType.DMA,
            pltpu.SemaphoreType.DMA,
        ],
    )(y, ei)


def _norm_from_deg(d_ref):
    deg = d_ref[0] + d_ref[1]
    return lax.rsqrt(jnp.maximum(deg, 1.0))[:, None]


def _pre_body(x_ref, w_ref, d_ref, o_ref):
    norm = _norm_from_deg(d_ref)
    xw = jnp.dot(x_ref[...], w_ref[...], preferred_element_type=jnp.float32)
    o_ref[...] = xw * norm


def _mid_body(p_ref, w_ref, b_ref, d_ref, o_ref):
    norm = _norm_from_deg(d_ref)
    b = b_ref[0:1, :]
    h = jnp.maximum((p_ref[0] + p_ref[1]) * norm + b, 0.0)
    hw = jnp.dot(h, w_ref[...], preferred_element_type=jnp.float32)
    o_ref[...] = hw * norm


def _post_body(p_ref, b_ref, d_ref, o_ref):
    norm = _norm_from_deg(d_ref)
    b = b_ref[0:1, :]
    o_ref[...] = jnp.maximum((p_ref[0] + p_ref[1]) * norm + b, 0.0)


def _row_spec():
    return pl.BlockSpec((ROW_BLK, D), lambda i: (i, 0))


def _part_spec():
    return pl.BlockSpec((2, ROW_BLK, D), lambda i: (0, i, 0))


def _deg_spec():
    return pl.BlockSpec((2, ROW_BLK), lambda i: (0, i))


def _full_spec(shape):
    return pl.BlockSpec(shape, lambda i: tuple(0 for _ in shape))


def _tc_pre(x, w, deg3):
    return pl.pallas_call(
        _pre_body,
        grid=(N_ROW_BLKS,),
        in_specs=[_row_spec(), _full_spec((D, D)), _deg_spec()],
        out_specs=_row_spec(),
        out_shape=jax.ShapeDtypeStruct((N_NODES, D), jnp.float32),
    )(x, w, deg3)


def _tc_mid(p, w, b8, deg3):
    return pl.pallas_call(
        _mid_body,
        grid=(N_ROW_BLKS,),
        in_specs=[_part_spec(), _full_spec((D, D)), _full_spec((8, D)),
                  _deg_spec()],
        out_specs=_row_spec(),
        out_shape=jax.ShapeDtypeStruct((N_NODES, D), jnp.float32),
    )(p, w, b8, deg3)


def _tc_post(p, b8, deg3):
    return pl.pallas_call(
        _post_body,
        grid=(N_ROW_BLKS,),
        in_specs=[_part_spec(), _full_spec((8, D)), _deg_spec()],
        out_specs=_row_spec(),
        out_shape=jax.ShapeDtypeStruct((N_NODES, D), jnp.float32),
    )(p, b8, deg3)


def kernel(features, edge_index, W1, b1, W2, b2):
    ei = edge_index.reshape(-1)
    b1_8 = jnp.broadcast_to(b1[None, :], (8, D))
    b2_8 = jnp.broadcast_to(b2[None, :], (8, D))

    deg_p = _degree_partials(ei)

    y1 = _tc_pre(features, W1, deg_p)
    agg1 = _aggregate_partials(y1, ei)
    y2 = _tc_mid(agg1, W2, b1_8, deg_p)
    agg2 = _aggregate_partials(y2, ei)
    return _tc_post(agg2, b2_8, deg_p)
